# baseline (speedup 1.0000x reference)
import functools

import jax
import jax.numpy as jnp
from jax import lax
from jax.experimental import pallas as pl
from jax.experimental.pallas import tpu as pltpu
from jax.experimental.pallas import tpu_sc as plsc

N_NODES = 10000
N_EDGES = 320000
HD = 128
BD = 16

NC = 2
NS = 16
NW = NC * NS

E_CHUNK = 40
N_CHUNKS = N_EDGES // E_CHUNK
CHUNKS_PER_W = N_CHUNKS // NW
PAIRS = CHUNKS_PER_W // 2

ZROWS = 80
N_ZCHUNKS = N_NODES // ZROWS
ZCHUNKS_PER_SUB = -(-N_ZCHUNKS // NS)

D_ROWS = 640


import numpy as np

_PERM = np.empty((HD,), dtype=np.int32)
for _k in range(HD // 32):
    for _l in range(16):
        _PERM[32 * _k + 2 * _l] = 32 * _k + _l
        _PERM[32 * _k + 2 * _l + 1] = 32 * _k + 16 + _l


def _abc_body(bond_ref, wc_ref, b1_ref, h_ref, wab_ref, c_ref, a_ref, b_ref):
    c_ref[...] = (
        jnp.dot(bond_ref[...], wc_ref[...], preferred_element_type=jnp.float32)
        + b1_ref[...]
    ).astype(jnp.bfloat16)

    @pl.when(pl.program_id(0) < 5)
    def _():
        hb = h_ref[...]
        a_ref[...] = jnp.dot(
            hb, wab_ref[0:HD, :], preferred_element_type=jnp.float32)
        b_ref[...] = jnp.dot(
            hb, wab_ref[HD:2 * HD, :], preferred_element_type=jnp.float32)


def _fin_body(s_ref, deg_ref, w_ref, b2_ref, o_ref):
    s = s_ref[0] + s_ref[1]
    deg = jnp.sum(deg_ref[...], axis=1)
    o_ref[...] = (
        jnp.dot(s, w_ref[...], preferred_element_type=jnp.float32)
        + deg[:, None] * b2_ref[...]
    )


def _sc_body(a_hbm, b_hbm, c_hbm, row_hbm, col_hbm, s_out, d_out,
             s_sh, idx_r0, idx_c0, idx_r1, idx_c1, idx_s0, idx_s1,
             a0, b0, c0, a1, b1, c1, urows, dloc,
             sem_a0, sem_b0, sem_c0, sem_a1, sem_b1, sem_c1,
             sem_s0, sem_s1):
    cid = lax.axis_index("c")
    sid = lax.axis_index("s")
    wid = sid * NC + cid

    zero16 = jnp.zeros((16,), jnp.float32)
    lanes = lax.broadcasted_iota(jnp.int32, (16,), 0)

    def zrow(i, carry):
        for g in range(HD // 16):
            urows[i, pl.ds(g * 16, 16)] = zero16
        return carry

    lax.fori_loop(0, E_CHUNK, zrow, 0)

    def drow(i, carry):
        dloc[i, pl.ds(0, 16)] = zero16
        return carry

    lax.fori_loop(0, D_ROWS, drow, 0)

    for j in range(ZCHUNKS_PER_SUB):
        zc = sid + j * NS

        @pl.when(zc < N_ZCHUNKS)
        def _():
            for h in range(ZROWS // E_CHUNK):
                pltpu.sync_copy(
                    urows,
                    s_sh.at[pl.ds(zc * ZROWS + h * E_CHUNK, E_CHUNK)])

    plsc.subcore_barrier()

    def start_gathers(m, i_r, i_c, abuf, bbuf, cbuf, s_a, s_b, s_c):
        base = (wid + m * NW) * E_CHUNK
        pltpu.sync_copy(row_hbm.at[pl.ds(base, E_CHUNK)], i_r)
        pltpu.sync_copy(col_hbm.at[pl.ds(base, E_CHUNK)], i_c)
        pltpu.async_copy(a_hbm.at[i_r], abuf, s_a)
        pltpu.async_copy(b_hbm.at[i_c], bbuf, s_b)
        pltpu.async_copy(c_hbm.at[pl.ds(base, E_CHUNK)], cbuf, s_c)

    def wait_gathers(i_r, i_c, abuf, bbuf, cbuf, s_a, s_b, s_c):
        pltpu.make_async_copy(a_hbm.at[i_r], abuf, s_a).wait()
        pltpu.make_async_copy(b_hbm.at[i_c], bbuf, s_b).wait()
        pltpu.make_async_copy(c_hbm.at[pl.ds(0, E_CHUNK)], cbuf, s_c).wait()

    def deg_hist(i_r):
        for off, lo in ((0, 0), (16, 0), (24, 8)):
            iv = i_r[pl.ds(off, 16)]
            for l in range(lo, 16):
                s = iv[l]
                iq = lax.shift_right_logical(s, 4)
                m = s & 15
                hot = jnp.where(lanes == m, jnp.float32(1.0), jnp.float32(0.0))
                dloc[iq, pl.ds(0, 16)] = dloc[iq, pl.ds(0, 16)] + hot

    mask_hi = jnp.int32(-65536)

    def silu_rows(abuf, bbuf, cbuf):
        def erow(i, c2):
            for k in range(HD // 32):
                wc = cbuf[i, pl.ds(16 * k, 16)]
                for h in range(2):
                    if h == 0:
                        cv = plsc.bitcast(wc << 16, jnp.float32)
                    else:
                        cv = plsc.bitcast(wc & mask_hi, jnp.float32)
                    sl = pl.ds(32 * k + 16 * h, 16)
                    t = abuf[i, sl] + bbuf[i, sl] + cv
                    urows[i, sl] = t / (jnp.float32(1.0) + jnp.exp(-t))
            return c2

        lax.fori_loop(0, E_CHUNK, erow, 0)

    def start_scatter(i_r, i_s, s_s):
        for off in (0, 16, 24):
            i_s[pl.ds(off, 16)] = i_r[pl.ds(off, 16)]
        pltpu.async_copy(urows, s_sh.at[i_s], s_s, add=True)

    def wait_scatter(i_s, s_s):
        pltpu.make_async_copy(urows, s_sh.at[i_s], s_s).wait()

    start_gathers(0, idx_r0, idx_c0, a0, b0, c0, sem_a0, sem_b0, sem_c0)

    def pair_body(j, carry):
        m0 = 2 * j

        @pl.when(j > 0)
        def _():
            wait_scatter(idx_s1, sem_s1)

        start_gathers(m0 + 1, idx_r1, idx_c1, a1, b1, c1,
                      sem_a1, sem_b1, sem_c1)
        deg_hist(idx_r0)
        deg_hist(idx_r1)
        wait_gathers(idx_r0, idx_c0, a0, b0, c0, sem_a0, sem_b0, sem_c0)
        silu_rows(a0, b0, c0)
        start_scatter(idx_r0, idx_s0, sem_s0)

        @pl.when(j < PAIRS - 1)
        def _():
            start_gathers(m0 + 2, idx_r0, idx_c0, a0, b0, c0,
                          sem_a0, sem_b0, sem_c0)

        wait_gathers(idx_r1, idx_c1, a1, b1, c1, sem_a1, sem_b1, sem_c1)
        wait_scatter(idx_s0, sem_s0)
        silu_rows(a1, b1, c1)
        start_scatter(idx_r1, idx_s1, sem_s1)

        return carry

    lax.fori_loop(0, PAIRS, pair_body, 0)
    wait_scatter(idx_s1, sem_s1)

    pltpu.sync_copy(dloc, d_out.at[wid])

    plsc.subcore_barrier()

    for j in range(-(-(N_NODES // E_CHUNK) // NS)):
        zc = sid + j * NS

        @pl.when(zc < N_NODES // E_CHUNK)
        def _():
            pltpu.sync_copy(s_sh.at[pl.ds(zc * E_CHUNK, E_CHUNK)], urows)
            pltpu.sync_copy(
                urows, s_out.at[cid].at[pl.ds(zc * E_CHUNK, E_CHUNK)])


@functools.lru_cache(maxsize=None)
def _make_sc_edges():
  return pl.kernel(
    _sc_body,
    out_type=(
        jax.ShapeDtypeStruct((NC, N_NODES, HD), jnp.float32),
        jax.ShapeDtypeStruct((NW, D_ROWS, 16), jnp.float32),
    ),
    mesh=plsc.VectorSubcoreMesh(
        core_axis_name="c", subcore_axis_name="s",
        num_cores=NC, num_subcores=NS),
    compiler_params=pltpu.CompilerParams(
        use_tc_tiling_on_sc=False, needs_layout_passes=False),
    scratch_types=(
        [pltpu.VMEM_SHARED((N_NODES, HD), jnp.float32)]
        + [pltpu.VMEM((E_CHUNK,), jnp.int32)] * 6
        + [pltpu.VMEM((E_CHUNK, HD), jnp.float32),
           pltpu.VMEM((E_CHUNK, HD), jnp.float32),
           pltpu.VMEM((E_CHUNK, HD // 2), jnp.int32),
           pltpu.VMEM((E_CHUNK, HD), jnp.float32),
           pltpu.VMEM((E_CHUNK, HD), jnp.float32),
           pltpu.VMEM((E_CHUNK, HD // 2), jnp.int32),
           pltpu.VMEM((E_CHUNK, HD), jnp.float32)]
        + [pltpu.VMEM((D_ROWS, 16), jnp.float32)]
        + [pltpu.SemaphoreType.DMA] * 8
    ),
  )


def _clip4(i):
    return (jnp.minimum(i, 4), 0)


_abc_call = pl.pallas_call(
    _abc_body,
    grid=(100,),
    in_specs=[
        pl.BlockSpec((3200, BD), lambda i: (i, 0)),
        pl.BlockSpec((BD, HD), lambda i: (0, 0)),
        pl.BlockSpec((1, HD), lambda i: (0, 0)),
        pl.BlockSpec((2000, HD), _clip4),
        pl.BlockSpec((2 * HD, HD), lambda i: (0, 0)),
    ],
    out_specs=[
        pl.BlockSpec((3200, HD), lambda i: (i, 0)),
        pl.BlockSpec((2000, HD), _clip4),
        pl.BlockSpec((2000, HD), _clip4),
    ],
    out_shape=[
        jax.ShapeDtypeStruct((N_EDGES, HD), jnp.bfloat16),
        jax.ShapeDtypeStruct((N_NODES, HD), jnp.float32),
        jax.ShapeDtypeStruct((N_NODES, HD), jnp.float32),
    ],
)

_fin_call = pl.pallas_call(
    _fin_body,
    grid=(5,),
    in_specs=[
        pl.BlockSpec((NC, 2000, HD), lambda i: (0, i, 0)),
        pl.BlockSpec((2000, NW), lambda i: (i, 0)),
        pl.BlockSpec((HD, HD), lambda i: (0, 0)),
        pl.BlockSpec((1, HD), lambda i: (0, 0)),
    ],
    out_specs=pl.BlockSpec((2000, HD), lambda i: (i, 0)),
    out_shape=jax.ShapeDtypeStruct((N_NODES, HD), jnp.float32),
)


@jax.jit
def kernel(h, edge_index, bond_features, W1, b1, W2, b2):
    row = edge_index[0].astype(jnp.int32)
    col = edge_index[1].astype(jnp.int32)

    C, A, B = _abc_call(
        bond_features, W1[2 * HD:][:, _PERM], b1[_PERM].reshape(1, HD),
        h, W1[: 2 * HD])
    C = lax.bitcast_convert_type(
        C.reshape(N_EDGES, HD // 2, 2), jnp.int32)

    S, D = _make_sc_edges()(A, B, C, row, col)

    deg = D.reshape(NW, D_ROWS * 16)[:, :N_NODES].T

    return _fin_call(S, deg, W2, b2.reshape(1, HD))


# speedup vs baseline: 3.2103x; 3.2103x over previous
import functools

import jax
import jax.numpy as jnp
from jax import lax
from jax.experimental import pallas as pl
from jax.experimental.pallas import tpu as pltpu
from jax.experimental.pallas import tpu_sc as plsc

N_NODES = 10000
N_EDGES = 320000
HD = 128
BD = 16

NC = 2
NS = 16
NW = NC * NS

E_CHUNK = 40
N_CHUNKS = N_EDGES // E_CHUNK
CHUNKS_PER_W = N_CHUNKS // NW
PAIRS = CHUNKS_PER_W // 2

ZROWS = 80
N_ZCHUNKS = N_NODES // ZROWS
ZCHUNKS_PER_SUB = -(-N_ZCHUNKS // NS)

D_ROWS = 640


import numpy as np

_PERM = np.empty((HD,), dtype=np.int32)
for _k in range(HD // 32):
    for _l in range(16):
        _PERM[32 * _k + 2 * _l] = 32 * _k + _l
        _PERM[32 * _k + 2 * _l + 1] = 32 * _k + 16 + _l


def _abc_body(bond_ref, wc_ref, b1_ref, h_ref, wab_ref, c_ref, a_ref, b_ref):
    c_ref[...] = (
        jnp.dot(bond_ref[...], wc_ref[...], preferred_element_type=jnp.float32)
        + b1_ref[...]
    )

    @pl.when(pl.program_id(0) < 5)
    def _():
        hb = h_ref[...]
        a_ref[...] = jnp.dot(
            hb, wab_ref[0:HD, :], preferred_element_type=jnp.float32)
        b_ref[...] = jnp.dot(
            hb, wab_ref[HD:2 * HD, :], preferred_element_type=jnp.float32)


def _fin_body(s_ref, deg_ref, w_ref, b2_ref, o_ref):
    s = s_ref[0] + s_ref[1]
    deg = jnp.sum(deg_ref[...], axis=1)
    o_ref[...] = (
        jnp.dot(s, w_ref[...], preferred_element_type=jnp.float32)
        + deg[:, None] * b2_ref[...]
    )


def _sc_body(a_hbm, b_hbm, c_hbm, row_hbm, col_hbm, s_out, d_out,
             s_sh, idx_r0, idx_c0, idx_r1, idx_c1, idx_s0, idx_s1,
             a0, b0, c0, a1, b1, c1, urows, dloc,
             sem_a0, sem_b0, sem_c0, sem_a1, sem_b1, sem_c1,
             sem_s0, sem_s1):
    cid = lax.axis_index("c")
    sid = lax.axis_index("s")
    wid = sid * NC + cid

    zero16 = jnp.zeros((16,), jnp.float32)
    lanes = lax.broadcasted_iota(jnp.int32, (16,), 0)

    def zrow(i, carry):
        for g in range(HD // 16):
            urows[i, pl.ds(g * 16, 16)] = zero16
        return carry

    lax.fori_loop(0, E_CHUNK, zrow, 0)

    def drow(i, carry):
        dloc[i, pl.ds(0, 16)] = zero16
        return carry

    lax.fori_loop(0, D_ROWS, drow, 0)

    for j in range(ZCHUNKS_PER_SUB):
        zc = sid + j * NS

        @pl.when(zc < N_ZCHUNKS)
        def _():
            for h in range(ZROWS // E_CHUNK):
                pltpu.sync_copy(
                    urows,
                    s_sh.at[pl.ds(zc * ZROWS + h * E_CHUNK, E_CHUNK)])

    plsc.subcore_barrier()

    def start_gathers(m, i_r, i_c, abuf, bbuf, cbuf, s_a, s_b, s_c):
        base = (wid + m * NW) * E_CHUNK
        pltpu.sync_copy(row_hbm.at[pl.ds(base, E_CHUNK)], i_r)
        pltpu.sync_copy(col_hbm.at[pl.ds(base, E_CHUNK)], i_c)
        pltpu.async_copy(a_hbm.at[i_r], abuf, s_a)
        pltpu.async_copy(b_hbm.at[i_c], bbuf, s_b)
        pltpu.async_copy(c_hbm.at[pl.ds(base, E_CHUNK)], cbuf, s_c)

    def wait_gathers(i_r, i_c, abuf, bbuf, cbuf, s_a, s_b, s_c):
        pltpu.make_async_copy(a_hbm.at[i_r], abuf, s_a).wait()
        pltpu.make_async_copy(b_hbm.at[i_c], bbuf, s_b).wait()
        pltpu.make_async_copy(c_hbm.at[pl.ds(0, E_CHUNK)], cbuf, s_c).wait()

    def deg_hist(i_r):
        for off, lo in ((0, 0), (16, 0), (24, 8)):
            iv = i_r[pl.ds(off, 16)]
            for l in range(lo, 16):
                s = iv[l]
                iq = lax.shift_right_logical(s, 4)
                m = s & 15
                hot = jnp.where(lanes == m, jnp.float32(1.0), jnp.float32(0.0))
                dloc[iq, pl.ds(0, 16)] = dloc[iq, pl.ds(0, 16)] + hot

    def silu_rows(abuf, bbuf, cbuf):
        def erow(i, c2):
            for g in range(HD // 16):
                sl = pl.ds(g * 16, 16)
                t = abuf[i, sl] + bbuf[i, sl] + cbuf[i, sl]
                urows[i, sl] = t / (jnp.float32(1.0) + jnp.exp(-t))
            return c2

        lax.fori_loop(0, E_CHUNK, erow, 0)

    def start_scatter(i_r, i_s, s_s):
        for off in (0, 16, 24):
            i_s[pl.ds(off, 16)] = i_r[pl.ds(off, 16)]
        pltpu.async_copy(urows, s_sh.at[i_s], s_s, add=True)

    def wait_scatter(i_s, s_s):
        pltpu.make_async_copy(urows, s_sh.at[i_s], s_s).wait()

    start_gathers(0, idx_r0, idx_c0, a0, b0, c0, sem_a0, sem_b0, sem_c0)

    def pair_body(j, carry):
        m0 = 2 * j

        @pl.when(j > 0)
        def _():
            wait_scatter(idx_s1, sem_s1)

        start_gathers(m0 + 1, idx_r1, idx_c1, a1, b1, c1,
                      sem_a1, sem_b1, sem_c1)
        deg_hist(idx_r0)
        deg_hist(idx_r1)
        wait_gathers(idx_r0, idx_c0, a0, b0, c0, sem_a0, sem_b0, sem_c0)
        silu_rows(a0, b0, c0)
        start_scatter(idx_r0, idx_s0, sem_s0)

        @pl.when(j < PAIRS - 1)
        def _():
            start_gathers(m0 + 2, idx_r0, idx_c0, a0, b0, c0,
                          sem_a0, sem_b0, sem_c0)

        wait_gathers(idx_r1, idx_c1, a1, b1, c1, sem_a1, sem_b1, sem_c1)
        wait_scatter(idx_s0, sem_s0)
        silu_rows(a1, b1, c1)
        start_scatter(idx_r1, idx_s1, sem_s1)

        return carry

    lax.fori_loop(0, PAIRS, pair_body, 0)
    wait_scatter(idx_s1, sem_s1)

    pltpu.sync_copy(dloc, d_out.at[wid])

    plsc.subcore_barrier()

    for j in range(-(-(N_NODES // E_CHUNK) // NS)):
        zc = sid + j * NS

        @pl.when(zc < N_NODES // E_CHUNK)
        def _():
            pltpu.sync_copy(s_sh.at[pl.ds(zc * E_CHUNK, E_CHUNK)], urows)
            pltpu.sync_copy(
                urows, s_out.at[cid].at[pl.ds(zc * E_CHUNK, E_CHUNK)])


@functools.lru_cache(maxsize=None)
def _make_sc_edges():
  return pl.kernel(
    _sc_body,
    out_type=(
        jax.ShapeDtypeStruct((NC, N_NODES, HD), jnp.float32),
        jax.ShapeDtypeStruct((NW, D_ROWS, 16), jnp.float32),
    ),
    mesh=plsc.VectorSubcoreMesh(
        core_axis_name="c", subcore_axis_name="s",
        num_cores=NC, num_subcores=NS),
    compiler_params=pltpu.CompilerParams(
        use_tc_tiling_on_sc=False, needs_layout_passes=False),
    scratch_types=(
        [pltpu.VMEM_SHARED((N_NODES, HD), jnp.float32)]
        + [pltpu.VMEM((E_CHUNK,), jnp.int32)] * 6
        + [pltpu.VMEM((E_CHUNK, HD), jnp.float32)] * 7
        + [pltpu.VMEM((D_ROWS, 16), jnp.float32)]
        + [pltpu.SemaphoreType.DMA] * 8
    ),
  )


def _clip4(i):
    return (jnp.minimum(i, 4), 0)


_abc_call = pl.pallas_call(
    _abc_body,
    grid=(100,),
    in_specs=[
        pl.BlockSpec((3200, BD), lambda i: (i, 0)),
        pl.BlockSpec((BD, HD), lambda i: (0, 0)),
        pl.BlockSpec((1, HD), lambda i: (0, 0)),
        pl.BlockSpec((2000, HD), _clip4),
        pl.BlockSpec((2 * HD, HD), lambda i: (0, 0)),
    ],
    out_specs=[
        pl.BlockSpec((3200, HD), lambda i: (i, 0)),
        pl.BlockSpec((2000, HD), _clip4),
        pl.BlockSpec((2000, HD), _clip4),
    ],
    out_shape=[
        jax.ShapeDtypeStruct((N_EDGES, HD), jnp.float32),
        jax.ShapeDtypeStruct((N_NODES, HD), jnp.float32),
        jax.ShapeDtypeStruct((N_NODES, HD), jnp.float32),
    ],
)

_fin_call = pl.pallas_call(
    _fin_body,
    grid=(5,),
    in_specs=[
        pl.BlockSpec((NC, 2000, HD), lambda i: (0, i, 0)),
        pl.BlockSpec((2000, NW), lambda i: (i, 0)),
        pl.BlockSpec((HD, HD), lambda i: (0, 0)),
        pl.BlockSpec((1, HD), lambda i: (0, 0)),
    ],
    out_specs=pl.BlockSpec((2000, HD), lambda i: (i, 0)),
    out_shape=jax.ShapeDtypeStruct((N_NODES, HD), jnp.float32),
)


@jax.jit
def kernel(h, edge_index, bond_features, W1, b1, W2, b2):
    row = edge_index[0].astype(jnp.int32)
    col = edge_index[1].astype(jnp.int32)

    C, A, B = _abc_call(
        bond_features, W1[2 * HD:], b1.reshape(1, HD), h, W1[: 2 * HD])

    S, D = _make_sc_edges()(A, B, C, row, col)

    deg = D.reshape(NW, D_ROWS * 16)[:, :N_NODES].T

    return _fin_call(S, deg, W2, b2.reshape(1, HD))


# speedup vs baseline: 3.7588x; 1.1709x over previous
import functools

import jax
import jax.numpy as jnp
from jax import lax
from jax.experimental import pallas as pl
from jax.experimental.pallas import tpu as pltpu
from jax.experimental.pallas import tpu_sc as plsc

N_NODES = 10000
N_EDGES = 320000
HD = 128
BD = 16

NC = 2
NS = 16
NW = NC * NS

E_CHUNK = 40
N_CHUNKS = N_EDGES // E_CHUNK
CHUNKS_PER_W = N_CHUNKS // NW
PAIRS = CHUNKS_PER_W // 2

ZROWS = 80
N_ZCHUNKS = N_NODES // ZROWS
ZCHUNKS_PER_SUB = -(-N_ZCHUNKS // NS)

D_ROWS = 640


import numpy as np

_PERM = np.empty((HD,), dtype=np.int32)
for _k in range(HD // 32):
    for _l in range(16):
        _PERM[32 * _k + 2 * _l] = 32 * _k + _l
        _PERM[32 * _k + 2 * _l + 1] = 32 * _k + 16 + _l


def _abc_body(bond_ref, wc_ref, b1_ref, h_ref, wab_ref, c_ref, a_ref, b_ref):
    c_ref[...] = (
        jnp.dot(bond_ref[...], wc_ref[...], preferred_element_type=jnp.float32)
        + b1_ref[...]
    )

    @pl.when(pl.program_id(0) < 5)
    def _():
        hb = h_ref[...]
        a_ref[...] = jnp.dot(
            hb, wab_ref[0:HD, :], preferred_element_type=jnp.float32)
        b_ref[...] = jnp.dot(
            hb, wab_ref[HD:2 * HD, :], preferred_element_type=jnp.float32)


def _fin_body(s_ref, deg_ref, w_ref, b2_ref, o_ref):
    s = s_ref[0] + s_ref[1]
    deg = jnp.sum(deg_ref[...], axis=1)
    o_ref[...] = (
        jnp.dot(s, w_ref[...], preferred_element_type=jnp.float32)
        + deg[:, None] * b2_ref[...]
    )


def _sc_body(a_hbm, b_hbm, c_hbm, ei_hbm, s_out, d_out,
             s_sh, idx0, idx1, idx_s0, idx_s1,
             a0, b0, c0, a1, b1, c1, urows, dloc,
             sem_a0, sem_b0, sem_c0, sem_a1, sem_b1, sem_c1,
             sem_s0, sem_s1):
    cid = lax.axis_index("c")
    sid = lax.axis_index("s")
    wid = sid * NC + cid

    zero16 = jnp.zeros((16,), jnp.float32)
    lanes = lax.broadcasted_iota(jnp.int32, (16,), 0)

    def zrow(i, carry):
        for g in range(HD // 16):
            urows[i, pl.ds(g * 16, 16)] = zero16
        return carry

    lax.fori_loop(0, E_CHUNK, zrow, 0)

    def drow(i, carry):
        dloc[i, pl.ds(0, 16)] = zero16
        return carry

    lax.fori_loop(0, D_ROWS, drow, 0)

    for j in range(ZCHUNKS_PER_SUB):
        zc = sid + j * NS

        @pl.when(zc < N_ZCHUNKS)
        def _():
            for h in range(ZROWS // E_CHUNK):
                pltpu.sync_copy(
                    urows,
                    s_sh.at[pl.ds(zc * ZROWS + h * E_CHUNK, E_CHUNK)])

    plsc.subcore_barrier()

    def start_gathers(m, ib, abuf, bbuf, cbuf, s_a, s_b, s_c):
        base = (wid + m * NW) * E_CHUNK
        pltpu.sync_copy(ei_hbm.at[:, pl.ds(base, E_CHUNK)], ib)
        pltpu.async_copy(a_hbm.at[ib.at[0]], abuf, s_a)
        pltpu.async_copy(b_hbm.at[ib.at[1]], bbuf, s_b)
        pltpu.async_copy(c_hbm.at[pl.ds(base, E_CHUNK)], cbuf, s_c)

    def wait_gathers(ib, abuf, bbuf, cbuf, s_a, s_b, s_c):
        pltpu.make_async_copy(a_hbm.at[ib.at[0]], abuf, s_a).wait()
        pltpu.make_async_copy(b_hbm.at[ib.at[1]], bbuf, s_b).wait()
        pltpu.make_async_copy(c_hbm.at[pl.ds(0, E_CHUNK)], cbuf, s_c).wait()

    def deg_hist(ib):
        for off, lo in ((0, 0), (16, 0), (24, 8)):
            iv = ib[0, pl.ds(off, 16)]
            for l in range(lo, 16):
                s = iv[l]
                iq = lax.shift_right_logical(s, 4)
                m = s & 15
                hot = jnp.where(lanes == m, jnp.float32(1.0), jnp.float32(0.0))
                dloc[iq, pl.ds(0, 16)] = dloc[iq, pl.ds(0, 16)] + hot

    def silu_rows(abuf, bbuf, cbuf):
        def erow(i, c2):
            for g in range(HD // 16):
                sl = pl.ds(g * 16, 16)
                t = abuf[i, sl] + bbuf[i, sl] + cbuf[i, sl]
                urows[i, sl] = t / (jnp.float32(1.0) + jnp.exp(-t))
            return c2

        lax.fori_loop(0, E_CHUNK, erow, 0)

    def start_scatter(ib, i_s, s_s):
        for off in (0, 16, 24):
            i_s[pl.ds(off, 16)] = ib[0, pl.ds(off, 16)]
        pltpu.async_copy(urows, s_sh.at[i_s], s_s, add=True)

    def wait_scatter(i_s, s_s):
        pltpu.make_async_copy(urows, s_sh.at[i_s], s_s).wait()

    start_gathers(0, idx0, a0, b0, c0, sem_a0, sem_b0, sem_c0)

    def pair_body(j, carry):
        m0 = 2 * j

        @pl.when(j > 0)
        def _():
            wait_scatter(idx_s1, sem_s1)

        start_gathers(m0 + 1, idx1, a1, b1, c1,
                      sem_a1, sem_b1, sem_c1)
        deg_hist(idx0)
        deg_hist(idx1)
        wait_gathers(idx0, a0, b0, c0, sem_a0, sem_b0, sem_c0)
        silu_rows(a0, b0, c0)
        start_scatter(idx0, idx_s0, sem_s0)

        @pl.when(j < PAIRS - 1)
        def _():
            start_gathers(m0 + 2, idx0, a0, b0, c0,
                          sem_a0, sem_b0, sem_c0)

        wait_gathers(idx1, a1, b1, c1, sem_a1, sem_b1, sem_c1)
        wait_scatter(idx_s0, sem_s0)
        silu_rows(a1, b1, c1)
        start_scatter(idx1, idx_s1, sem_s1)

        return carry

    lax.fori_loop(0, PAIRS, pair_body, 0)
    wait_scatter(idx_s1, sem_s1)

    pltpu.sync_copy(dloc, d_out.at[wid])

    plsc.subcore_barrier()

    for j in range(-(-(N_NODES // E_CHUNK) // NS)):
        zc = sid + j * NS

        @pl.when(zc < N_NODES // E_CHUNK)
        def _():
            pltpu.sync_copy(s_sh.at[pl.ds(zc * E_CHUNK, E_CHUNK)], urows)
            pltpu.sync_copy(
                urows, s_out.at[cid].at[pl.ds(zc * E_CHUNK, E_CHUNK)])


@functools.lru_cache(maxsize=None)
def _make_sc_edges():
  return pl.kernel(
    _sc_body,
    out_type=(
        jax.ShapeDtypeStruct((NC, N_NODES, HD), jnp.float32),
        jax.ShapeDtypeStruct((NW, D_ROWS, 16), jnp.float32),
    ),
    mesh=plsc.VectorSubcoreMesh(
        core_axis_name="c", subcore_axis_name="s",
        num_cores=NC, num_subcores=NS),
    compiler_params=pltpu.CompilerParams(
        use_tc_tiling_on_sc=False, needs_layout_passes=False),
    scratch_types=(
        [pltpu.VMEM_SHARED((N_NODES, HD), jnp.float32)]
        + [pltpu.VMEM((2, E_CHUNK), jnp.int32)] * 2
        + [pltpu.VMEM((E_CHUNK,), jnp.int32)] * 2
        + [pltpu.VMEM((E_CHUNK, HD), jnp.float32)] * 7
        + [pltpu.VMEM((D_ROWS, 16), jnp.float32)]
        + [pltpu.SemaphoreType.DMA] * 8
    ),
  )


def _clip4(i):
    return (jnp.minimum(i, 4), 0)


_abc_call = pl.pallas_call(
    _abc_body,
    grid=(100,),
    in_specs=[
        pl.BlockSpec((3200, BD), lambda i: (i, 0)),
        pl.BlockSpec((BD, HD), lambda i: (0, 0)),
        pl.BlockSpec((1, HD), lambda i: (0, 0)),
        pl.BlockSpec((2000, HD), _clip4),
        pl.BlockSpec((2 * HD, HD), lambda i: (0, 0)),
    ],
    out_specs=[
        pl.BlockSpec((3200, HD), lambda i: (i, 0)),
        pl.BlockSpec((2000, HD), _clip4),
        pl.BlockSpec((2000, HD), _clip4),
    ],
    out_shape=[
        jax.ShapeDtypeStruct((N_EDGES, HD), jnp.float32),
        jax.ShapeDtypeStruct((N_NODES, HD), jnp.float32),
        jax.ShapeDtypeStruct((N_NODES, HD), jnp.float32),
    ],
)

_fin_call = pl.pallas_call(
    _fin_body,
    grid=(5,),
    in_specs=[
        pl.BlockSpec((NC, 2000, HD), lambda i: (0, i, 0)),
        pl.BlockSpec((2000, NW), lambda i: (i, 0)),
        pl.BlockSpec((HD, HD), lambda i: (0, 0)),
        pl.BlockSpec((1, HD), lambda i: (0, 0)),
    ],
    out_specs=pl.BlockSpec((2000, HD), lambda i: (i, 0)),
    out_shape=jax.ShapeDtypeStruct((N_NODES, HD), jnp.float32),
)


@jax.jit
def kernel(h, edge_index, bond_features, W1, b1, W2, b2):
    ei32 = edge_index.astype(jnp.int32)

    C, A, B = _abc_call(
        bond_features, W1[2 * HD:], b1.reshape(1, HD), h, W1[: 2 * HD])

    S, D = _make_sc_edges()(A, B, C, ei32)

    deg = D.reshape(NW, D_ROWS * 16)[:, :N_NODES].T

    return _fin_call(S, deg, W2, b2.reshape(1, HD))


# speedup vs baseline: 3.9185x; 1.0425x over previous
import functools

import jax
import jax.numpy as jnp
from jax import lax
from jax.experimental import pallas as pl
from jax.experimental.pallas import tpu as pltpu
from jax.experimental.pallas import tpu_sc as plsc

N_NODES = 10000
N_EDGES = 320000
HD = 128
BD = 16

NC = 2
NS = 16
NW = NC * NS

E_CHUNK = 40
N_CHUNKS = N_EDGES // E_CHUNK
CHUNKS_PER_W = N_CHUNKS // NW
PAIRS = CHUNKS_PER_W // 2

ZROWS = 80
N_ZCHUNKS = N_NODES // ZROWS
ZCHUNKS_PER_SUB = -(-N_ZCHUNKS // NS)

D_ROWS = 640


import numpy as np

_PERM = np.empty((HD,), dtype=np.int32)
for _k in range(HD // 32):
    for _l in range(16):
        _PERM[32 * _k + 2 * _l] = 32 * _k + _l
        _PERM[32 * _k + 2 * _l + 1] = 32 * _k + 16 + _l


def _abc_body(bond_ref, wc_ref, b1_ref, h_ref, wab_ref, c_ref, a_ref, b_ref):
    c_ref[...] = (
        jnp.dot(bond_ref[...], wc_ref[...], preferred_element_type=jnp.float32)
        + b1_ref[...]
    )

    @pl.when(pl.program_id(0) < 5)
    def _():
        hb = h_ref[...]
        a_ref[...] = jnp.dot(
            hb, wab_ref[0:HD, :], preferred_element_type=jnp.float32)
        b_ref[...] = jnp.dot(
            hb, wab_ref[HD:2 * HD, :], preferred_element_type=jnp.float32)


def _fin_body(s_ref, deg_ref, w_ref, b2_ref, o_ref):
    s = s_ref[0] + s_ref[1]
    deg = jnp.sum(deg_ref[...], axis=1)
    o_ref[...] = (
        jnp.dot(s, w_ref[...], preferred_element_type=jnp.float32)
        + deg[:, None] * b2_ref[...]
    )


def _sc_body(a_hbm, b_hbm, c_hbm, ei_hbm, s_out, d_out,
             s_sh, idxP, idx_s0, idx_s1,
             a0, b0, c0, a1, b1, c1, urows, dloc,
             sem_a0, sem_b0, sem_c0, sem_a1, sem_b1, sem_c1,
             sem_s0, sem_s1):
    cid = lax.axis_index("c")
    sid = lax.axis_index("s")
    wid = sid * NC + cid

    zero16 = jnp.zeros((16,), jnp.float32)
    lanes = lax.broadcasted_iota(jnp.int32, (16,), 0)

    def zrow(i, carry):
        for g in range(HD // 16):
            urows[i, pl.ds(g * 16, 16)] = zero16
        return carry

    lax.fori_loop(0, E_CHUNK, zrow, 0)

    def drow(i, carry):
        dloc[i, pl.ds(0, 16)] = zero16
        return carry

    lax.fori_loop(0, D_ROWS, drow, 0)

    for j in range(ZCHUNKS_PER_SUB):
        zc = sid + j * NS

        @pl.when(zc < N_ZCHUNKS)
        def _():
            for h in range(ZROWS // E_CHUNK):
                pltpu.sync_copy(
                    urows,
                    s_sh.at[pl.ds(zc * ZROWS + h * E_CHUNK, E_CHUNK)])

    plsc.subcore_barrier()

    wbase = wid * CHUNKS_PER_W * E_CHUNK

    def load_pair_idx(jp):
        pltpu.sync_copy(
            ei_hbm.at[:, pl.ds(wbase + 2 * jp * E_CHUNK, 2 * E_CHUNK)], idxP)

    def start_gathers(m, half, abuf, bbuf, cbuf, s_a, s_b, s_c):
        base = wbase + m * E_CHUNK
        isl = idxP.at[0, pl.ds(half * E_CHUNK, E_CHUNK)]
        csl = idxP.at[1, pl.ds(half * E_CHUNK, E_CHUNK)]
        pltpu.async_copy(a_hbm.at[isl], abuf, s_a)
        pltpu.async_copy(b_hbm.at[csl], bbuf, s_b)
        pltpu.async_copy(c_hbm.at[pl.ds(base, E_CHUNK)], cbuf, s_c)

    def wait_gathers(half, abuf, bbuf, cbuf, s_a, s_b, s_c):
        isl = idxP.at[0, pl.ds(half * E_CHUNK, E_CHUNK)]
        csl = idxP.at[1, pl.ds(half * E_CHUNK, E_CHUNK)]
        pltpu.make_async_copy(a_hbm.at[isl], abuf, s_a).wait()
        pltpu.make_async_copy(b_hbm.at[csl], bbuf, s_b).wait()
        pltpu.make_async_copy(c_hbm.at[pl.ds(0, E_CHUNK)], cbuf, s_c).wait()

    def deg_hist(half):
        for off, lo in ((0, 0), (16, 0), (24, 8)):
            iv = idxP[0, pl.ds(half * E_CHUNK + off, 16)]
            for l in range(lo, 16):
                s = iv[l]
                iq = lax.shift_right_logical(s, 4)
                m = s & 15
                hot = jnp.where(lanes == m, jnp.float32(1.0), jnp.float32(0.0))
                dloc[iq, pl.ds(0, 16)] = dloc[iq, pl.ds(0, 16)] + hot

    def silu_rows(abuf, bbuf, cbuf):
        def erow(i, c2):
            for g in range(HD // 16):
                sl = pl.ds(g * 16, 16)
                t = abuf[i, sl] + bbuf[i, sl] + cbuf[i, sl]
                urows[i, sl] = t / (jnp.float32(1.0) + jnp.exp(-t))
            return c2

        lax.fori_loop(0, E_CHUNK, erow, 0)

    def copy_scatter_idx(half, i_s):
        for off in (0, 16, 24):
            i_s[pl.ds(off, 16)] = idxP[0, pl.ds(half * E_CHUNK + off, 16)]

    def start_scatter(i_s, s_s):
        pltpu.async_copy(urows, s_sh.at[i_s], s_s, add=True)

    def wait_scatter(i_s, s_s):
        pltpu.make_async_copy(urows, s_sh.at[i_s], s_s).wait()

    load_pair_idx(0)
    start_gathers(0, 0, a0, b0, c0, sem_a0, sem_b0, sem_c0)

    def pair_body(j, carry):
        m0 = 2 * j

        @pl.when(j > 0)
        def _():
            wait_scatter(idx_s1, sem_s1)

        start_gathers(m0 + 1, 1, a1, b1, c1, sem_a1, sem_b1, sem_c1)
        deg_hist(0)
        deg_hist(1)
        copy_scatter_idx(0, idx_s0)
        copy_scatter_idx(1, idx_s1)
        wait_gathers(0, a0, b0, c0, sem_a0, sem_b0, sem_c0)
        silu_rows(a0, b0, c0)
        start_scatter(idx_s0, sem_s0)
        wait_gathers(1, a1, b1, c1, sem_a1, sem_b1, sem_c1)

        @pl.when(j < PAIRS - 1)
        def _():
            load_pair_idx(j + 1)
            start_gathers(m0 + 2, 0, a0, b0, c0,
                          sem_a0, sem_b0, sem_c0)

        wait_scatter(idx_s0, sem_s0)
        silu_rows(a1, b1, c1)
        start_scatter(idx_s1, sem_s1)

        return carry

    lax.fori_loop(0, PAIRS, pair_body, 0)
    wait_scatter(idx_s1, sem_s1)

    pltpu.sync_copy(dloc, d_out.at[wid])

    plsc.subcore_barrier()

    for j in range(-(-(N_NODES // E_CHUNK) // NS)):
        zc = sid + j * NS

        @pl.when(zc < N_NODES // E_CHUNK)
        def _():
            pltpu.sync_copy(s_sh.at[pl.ds(zc * E_CHUNK, E_CHUNK)], urows)
            pltpu.sync_copy(
                urows, s_out.at[cid].at[pl.ds(zc * E_CHUNK, E_CHUNK)])


@functools.lru_cache(maxsize=None)
def _make_sc_edges():
  return pl.kernel(
    _sc_body,
    out_type=(
        jax.ShapeDtypeStruct((NC, N_NODES, HD), jnp.float32),
        jax.ShapeDtypeStruct((NW, D_ROWS, 16), jnp.float32),
    ),
    mesh=plsc.VectorSubcoreMesh(
        core_axis_name="c", subcore_axis_name="s",
        num_cores=NC, num_subcores=NS),
    compiler_params=pltpu.CompilerParams(
        use_tc_tiling_on_sc=False, needs_layout_passes=False),
    scratch_types=(
        [pltpu.VMEM_SHARED((N_NODES, HD), jnp.float32)]
        + [pltpu.VMEM((2, 2 * E_CHUNK), jnp.int32)]
        + [pltpu.VMEM((E_CHUNK,), jnp.int32)] * 2
        + [pltpu.VMEM((E_CHUNK, HD), jnp.float32)] * 7
        + [pltpu.VMEM((D_ROWS, 16), jnp.float32)]
        + [pltpu.SemaphoreType.DMA] * 8
    ),
  )


def _clip4(i):
    return (jnp.minimum(i, 4), 0)


_abc_call = pl.pallas_call(
    _abc_body,
    grid=(100,),
    in_specs=[
        pl.BlockSpec((3200, BD), lambda i: (i, 0)),
        pl.BlockSpec((BD, HD), lambda i: (0, 0)),
        pl.BlockSpec((1, HD), lambda i: (0, 0)),
        pl.BlockSpec((2000, HD), _clip4),
        pl.BlockSpec((2 * HD, HD), lambda i: (0, 0)),
    ],
    out_specs=[
        pl.BlockSpec((3200, HD), lambda i: (i, 0)),
        pl.BlockSpec((2000, HD), _clip4),
        pl.BlockSpec((2000, HD), _clip4),
    ],
    out_shape=[
        jax.ShapeDtypeStruct((N_EDGES, HD), jnp.float32),
        jax.ShapeDtypeStruct((N_NODES, HD), jnp.float32),
        jax.ShapeDtypeStruct((N_NODES, HD), jnp.float32),
    ],
)

_fin_call = pl.pallas_call(
    _fin_body,
    grid=(5,),
    in_specs=[
        pl.BlockSpec((NC, 2000, HD), lambda i: (0, i, 0)),
        pl.BlockSpec((2000, NW), lambda i: (i, 0)),
        pl.BlockSpec((HD, HD), lambda i: (0, 0)),
        pl.BlockSpec((1, HD), lambda i: (0, 0)),
    ],
    out_specs=pl.BlockSpec((2000, HD), lambda i: (i, 0)),
    out_shape=jax.ShapeDtypeStruct((N_NODES, HD), jnp.float32),
)


@jax.jit
def kernel(h, edge_index, bond_features, W1, b1, W2, b2):
    ei32 = edge_index.astype(jnp.int32)

    C, A, B = _abc_call(
        bond_features, W1[2 * HD:], b1.reshape(1, HD), h, W1[: 2 * HD])

    S, D = _make_sc_edges()(A, B, C, ei32)

    deg = D.reshape(NW, D_ROWS * 16)[:, :N_NODES].T

    return _fin_call(S, deg, W2, b2.reshape(1, HD))
